# Initial kernel scaffold; baseline (speedup 1.0000x reference)
import jax
import jax.numpy as jnp
from jax.experimental import pallas as pl


def kernel(q, v, embeddings):
    raise NotImplementedError("write your pallas kernel here")


# speedup vs baseline: 8.1875x; 8.1875x over previous
import functools

import jax
import jax.numpy as jnp
from jax import lax
from jax.experimental import pallas as pl
from jax.experimental.pallas import tpu as pltpu
from jax.experimental.pallas import tpu_sc as plsc

_D = 32
_V = 257
_S = 2048
_PROWS = 2 * _S - 1
_LEAD = _S - _V // 2
_MID = _V - 2
_TRAIL = _PROWS - _LEAD - _MID

_NC = 2
_NS = 16
_ROWS_PER_W = _S // (_NC * _NS)
_FILL = _LEAD // _NS
_FIRE = 8


_WROWS = _S - 1 + _ROWS_PER_W


def _band_body(emb_hbm, out_hbm, p_sh, ev0, ev1, buf, mid_v, win, sem):
    c = lax.axis_index("c")
    s = lax.axis_index("s")

    pltpu.sync_copy(emb_hbm.at[pl.ds(0, _D)], ev0)
    pltpu.sync_copy(emb_hbm.at[pl.ds((_V - 1) * _D, _D)], ev1)

    def _fill(src_ref):
        a = src_ref[pl.ds(0, 16)]
        b = src_ref[pl.ds(16, 16)]

        def body(r, carry):
            buf[pl.ds(r * _D, 16)] = a
            buf[pl.ds(r * _D + 16, 16)] = b
            return carry

        lax.fori_loop(0, _FILL, body, 0)

    _fill(ev0)
    pltpu.sync_copy(buf, p_sh.at[pl.ds(s * _FILL * _D, _FILL * _D)])
    _fill(ev1)
    pltpu.sync_copy(
        buf, p_sh.at[pl.ds((_LEAD + _MID + s * _FILL) * _D, _FILL * _D)]
    )

    @pl.when(s == 0)
    def _():
        pltpu.sync_copy(emb_hbm.at[pl.ds(_D, _MID * _D)], mid_v)
        pltpu.sync_copy(mid_v, p_sh.at[pl.ds(_LEAD * _D, _MID * _D)])

    plsc.subcore_barrier()

    wid = s * _NC + c
    base = wid * _ROWS_PER_W
    row_words = _S * _D
    pltpu.sync_copy(
        p_sh.at[pl.ds((_S - _ROWS_PER_W - base) * _D, _WROWS * _D)], win
    )
    for ko in range(_ROWS_PER_W // _FIRE):
        handles = []
        for j in range(_FIRE):
            k = ko * _FIRE + j
            src = win.at[pl.ds((_ROWS_PER_W - 1 - k) * _D, row_words)]
            dst = out_hbm.at[pl.ds((base + k) * row_words, row_words)]
            handles.append(pltpu.async_copy(src, dst, sem))
        for h in handles:
            h.wait()


@jax.jit
def _expand(emb_flat):
    mesh = plsc.VectorSubcoreMesh(core_axis_name="c", subcore_axis_name="s")
    call = functools.partial(
        pl.kernel,
        out_type=jax.ShapeDtypeStruct((_S * _S * _D,), jnp.float32),
        mesh=mesh,
        scratch_types=[
            pltpu.VMEM_SHARED((_PROWS * _D,), jnp.float32),
            pltpu.VMEM((_D,), jnp.float32),
            pltpu.VMEM((_D,), jnp.float32),
            pltpu.VMEM((_FILL * _D,), jnp.float32),
            pltpu.VMEM((_MID * _D,), jnp.float32),
            pltpu.VMEM((_WROWS * _D,), jnp.float32),
            pltpu.SemaphoreType.DMA,
        ],
    )(_band_body)
    return call(emb_flat)


def kernel(q, v, embeddings):
    assert q.shape[1] == _S and v.shape[1] == _S
    assert embeddings.shape == (_V, _D)
    out_flat = _expand(embeddings.reshape(-1))
    return out_flat.reshape(_S, _S, _D)


# speedup vs baseline: 26.8223x; 3.2760x over previous
import functools

import jax
import jax.numpy as jnp
from jax import lax
from jax.experimental import pallas as pl
from jax.experimental.pallas import tpu as pltpu
from jax.experimental.pallas import tpu_sc as plsc

_D = 32
_V = 257
_S = 2048
_MAXP = (_V - 1) // 2

_NC = 2
_NS = 16
_ROWS_PER_W = _S // (_NC * _NS)
_UPD_CHUNKS = 17
_UPD_SPAN = _UPD_CHUNKS * 16


def _band_body(emb_hbm, out_hbm, embv, obuf, sem):
    c = lax.axis_index("c")
    s = lax.axis_index("s")
    base = (s * _NC + c) * _ROWS_PER_W

    pltpu.sync_copy(emb_hbm.at[pl.ds(0, _V - 1), :], embv.at[pl.ds(0, _V - 1), :])
    pltpu.sync_copy(emb_hbm.at[pl.ds(_V - 1, 1), :], embv.at[pl.ds(_V - 1, 1), :])

    lanes = lax.iota(jnp.int32, 16)

    def write_chunk(i, j0):
        pos = jnp.clip(j0 + lanes - i + _MAXP, 0, _V - 1)
        for d in range(_D):
            rows = plsc.load_gather(embv, [pos, jnp.full((16,), d, jnp.int32)])
            obuf[d, pl.ds(j0, 16)] = rows

    def full_chunk(ci, carry):
        write_chunk(base, ci * 16)
        return carry

    lax.fori_loop(0, _S // 16, full_chunk, 0)
    pltpu.async_copy(obuf, out_hbm.at[base], sem)

    def row(k, carry):
        i = base + k
        s0 = jnp.clip((i - _MAXP) & ~15, 0, _S - _UPD_SPAN)
        pltpu.make_async_copy(obuf, out_hbm.at[i], sem).wait()
        for u in range(_UPD_CHUNKS):
            write_chunk(i, s0 + u * 16)
        pltpu.async_copy(obuf, out_hbm.at[i], sem)
        return carry

    lax.fori_loop(1, _ROWS_PER_W, row, 0)
    pltpu.make_async_copy(obuf, out_hbm.at[base], sem).wait()


@jax.jit
def _expand(emb):
    mesh = plsc.VectorSubcoreMesh(core_axis_name="c", subcore_axis_name="s")
    call = functools.partial(
        pl.kernel,
        out_type=jax.ShapeDtypeStruct((_S, _D, _S), jnp.float32),
        mesh=mesh,
        compiler_params=pltpu.CompilerParams(needs_layout_passes=False),
        scratch_types=[
            pltpu.VMEM((_V, _D), jnp.float32),
            pltpu.VMEM((_D, _S), jnp.float32),
            pltpu.SemaphoreType.DMA,
        ],
    )(_band_body)
    return call(emb)


def kernel(q, v, embeddings):
    assert q.shape[1] == _S and v.shape[1] == _S
    assert embeddings.shape == (_V, _D)
    out_t = _expand(embeddings)
    return jnp.swapaxes(out_t, 1, 2)


# speedup vs baseline: 46.7369x; 1.7425x over previous
import functools

import jax
import jax.numpy as jnp
from jax import lax
from jax.experimental import pallas as pl
from jax.experimental.pallas import tpu as pltpu
from jax.experimental.pallas import tpu_sc as plsc

_D = 32
_V = 257
_S = 2048
_MAXP = (_V - 1) // 2

_NC = 2
_NS = 16
_ROWS_PER_W = _S // (_NC * _NS)
_UPD_CHUNKS = 17
_UPD_SPAN = _UPD_CHUNKS * 16


_EPAD = 33


def _band_body(emb_hbm, out_hbm, embv, epad, obuf, sem):
    c = lax.axis_index("c")
    s = lax.axis_index("s")
    base = (s * _NC + c) * _ROWS_PER_W

    pltpu.sync_copy(emb_hbm.at[pl.ds(0, _V - 1), :], embv.at[pl.ds(0, _V - 1), :])
    pltpu.sync_copy(emb_hbm.at[pl.ds(_V - 1, 1), :], embv.at[pl.ds(_V - 1, 1), :])

    lanes = lax.iota(jnp.int32, 16)

    def stage(pos, carry):
        a = plsc.load_gather(embv, [jnp.full((16,), pos, jnp.int32), lanes])
        b = plsc.load_gather(embv, [jnp.full((16,), pos, jnp.int32), lanes + 16])
        epad[pl.ds(pos * _EPAD, 16)] = a
        epad[pl.ds(pos * _EPAD + 16, 16)] = b
        return carry

    lax.fori_loop(0, _V, stage, 0)

    def write_chunk(i, j0):
        pos = jnp.clip(j0 + lanes - i + _MAXP, 0, _V - 1) * _EPAD
        for d in range(_D):
            rows = plsc.load_gather(epad, [pos + d])
            obuf[d, pl.ds(j0, 16)] = rows

    def full_chunk(ci, carry):
        write_chunk(base, ci * 16)
        return carry

    lax.fori_loop(0, _S // 16, full_chunk, 0)
    pltpu.async_copy(obuf, out_hbm.at[base], sem)

    def row(k, carry):
        i = base + k
        s0 = jnp.clip((i - _MAXP) & ~15, 0, _S - _UPD_SPAN)
        pltpu.make_async_copy(obuf, out_hbm.at[i], sem).wait()
        for u in range(_UPD_CHUNKS):
            write_chunk(i, s0 + u * 16)
        pltpu.async_copy(obuf, out_hbm.at[i], sem)
        return carry

    lax.fori_loop(1, _ROWS_PER_W, row, 0)
    pltpu.make_async_copy(obuf, out_hbm.at[base], sem).wait()


@jax.jit
def _expand(emb):
    mesh = plsc.VectorSubcoreMesh(core_axis_name="c", subcore_axis_name="s")
    call = functools.partial(
        pl.kernel,
        out_type=jax.ShapeDtypeStruct((_S, _D, _S), jnp.float32),
        mesh=mesh,
        compiler_params=pltpu.CompilerParams(needs_layout_passes=False),
        scratch_types=[
            pltpu.VMEM((_V, _D), jnp.float32),
            pltpu.VMEM((_V * _EPAD,), jnp.float32),
            pltpu.VMEM((_D, _S), jnp.float32),
            pltpu.SemaphoreType.DMA,
        ],
    )(_band_body)
    return call(emb)


def kernel(q, v, embeddings):
    assert q.shape[1] == _S and v.shape[1] == _S
    assert embeddings.shape == (_V, _D)
    out_t = _expand(embeddings)
    return jnp.swapaxes(out_t, 1, 2)


# speedup vs baseline: 58.2244x; 1.2458x over previous
import functools

import jax
import jax.numpy as jnp
from jax import lax
from jax.experimental import pallas as pl
from jax.experimental.pallas import tpu as pltpu
from jax.experimental.pallas import tpu_sc as plsc

_D = 32
_V = 257
_S = 2048
_MAXP = (_V - 1) // 2

_NC = 2
_NS = 16
_ROWS_PER_W = _S // (_NC * _NS)
_EPAD = 33
_STRIPE = 384
_NTILES = (_S - _STRIPE) // 128


def _band_body(emb_hbm, out_hbm, tmp, epad, bigbuf, sb0, sb1, sem_s, sem_c):
    c = lax.axis_index("c")
    s = lax.axis_index("s")
    base = (s * _NC + c) * _ROWS_PER_W

    pltpu.sync_copy(emb_hbm, tmp)
    lanes = lax.iota(jnp.int32, 16)

    def stage(pos, carry):
        epad[pl.ds(pos * _EPAD, 16)] = tmp[pl.ds(pos * _D, 16)]
        epad[pl.ds(pos * _EPAD + 16, 16)] = tmp[pl.ds(pos * _D + 16, 16)]
        return carry

    lax.fori_loop(0, _V, stage, 0)

    def gather_chunk(buf, i, j0, x0):
        pos = jnp.clip(j0 + lanes - i + _MAXP, 0, _V - 1) * _EPAD
        for d in range(_D):
            buf[d, pl.ds(x0, 16)] = plsc.load_gather(epad, [pos + d])

    def full_chunk(ci, carry):
        gather_chunk(bigbuf, base, ci * 16, ci * 16)
        return carry

    lax.fori_loop(0, _S // 16, full_chunk, 0)

    a_col = jnp.clip(((base - _MAXP) >> 7) << 7, 0, _S - _STRIPE)
    a_col = pl.multiple_of(a_col, 128)
    aidx = a_col >> 7

    def row(k, sb):
        i = base + k

        @pl.when(k >= 2)
        def _():
            pltpu.make_async_copy(
                sb, out_hbm.at[base, :, pl.ds(a_col, _STRIPE)], sem_s
            ).wait()

        def upd(u, carry):
            gather_chunk(sb, i, a_col + u * 16, u * 16)
            return carry

        lax.fori_loop(0, _STRIPE // 16, upd, 0)
        pltpu.async_copy(sb, out_hbm.at[i, :, pl.ds(a_col, _STRIPE)], sem_s)
        for t in range(_NTILES):
            @pl.when(t < aidx)
            def _():
                pltpu.async_copy(
                    bigbuf.at[:, pl.ds(128 * t, 128)],
                    out_hbm.at[i, :, pl.ds(128 * t, 128)],
                    sem_c,
                )

            @pl.when(t >= aidx)
            def _():
                pltpu.async_copy(
                    bigbuf.at[:, pl.ds(128 * t + _STRIPE, 128)],
                    out_hbm.at[i, :, pl.ds(128 * t + _STRIPE, 128)],
                    sem_c,
                )

        @pl.when(k >= 1)
        def _():
            for t in range(_NTILES):
                pltpu.make_async_copy(
                    bigbuf.at[:, pl.ds(0, 128)],
                    out_hbm.at[base, :, pl.ds(0, 128)],
                    sem_c,
                ).wait()

    def row_pair(k2, carry):
        row(2 * k2, sb0)
        row(2 * k2 + 1, sb1)
        return carry

    lax.fori_loop(0, _ROWS_PER_W // 2, row_pair, 0)

    for t in range(_NTILES):
        pltpu.make_async_copy(
            bigbuf.at[:, pl.ds(0, 128)],
            out_hbm.at[base, :, pl.ds(0, 128)],
            sem_c,
        ).wait()
    for sb in (sb0, sb1):
        pltpu.make_async_copy(
            sb, out_hbm.at[base, :, pl.ds(a_col, _STRIPE)], sem_s
        ).wait()


@jax.jit
def _expand(emb_flat):
    mesh = plsc.VectorSubcoreMesh(core_axis_name="c", subcore_axis_name="s")
    call = functools.partial(
        pl.kernel,
        out_type=jax.ShapeDtypeStruct((_S, _D, _S), jnp.float32),
        mesh=mesh,
        compiler_params=pltpu.CompilerParams(needs_layout_passes=False),
        scratch_types=[
            pltpu.VMEM((_V * _D,), jnp.float32),
            pltpu.VMEM((_V * _EPAD,), jnp.float32),
            pltpu.VMEM((_D, _S), jnp.float32),
            pltpu.VMEM((_D, _STRIPE), jnp.float32),
            pltpu.VMEM((_D, _STRIPE), jnp.float32),
            pltpu.SemaphoreType.DMA,
            pltpu.SemaphoreType.DMA,
        ],
    )(_band_body)
    return call(emb_flat)


def kernel(q, v, embeddings):
    assert q.shape[1] == _S and v.shape[1] == _S
    assert embeddings.shape == (_V, _D)
    out_t = _expand(embeddings.reshape(-1))
    return jnp.swapaxes(out_t, 1, 2)
